# baseline (speedup 1.0000x reference)
import functools

import jax
import jax.numpy as jnp
from jax import lax
from jax.experimental import pallas as pl
from jax.experimental.pallas import tpu as pltpu
from jax.experimental.pallas import tpu_sc as plsc

NC = 2
NS = 16
NGRAPHS = 64


@functools.cache
def _make_scatter(n, e, d):
    NW = NC * NS
    EPW = e // NW
    CH = 125
    NCH = EPW // CH
    NBUF = 8
    ROUNDS = NCH // NBUF
    assert EPW * NW == e and CH * NCH == EPW and NBUF * ROUNDS == NCH
    RPT = (n // NS) // 8 * 8
    REM = n - RPT * NS
    assert RPT % 8 == 0 and REM % 8 == 0

    mesh = plsc.VectorSubcoreMesh(
        core_axis_name="c", subcore_axis_name="s",
        num_cores=NC, num_subcores=NS)
    ZR = 208
    NZC = RPT // ZR
    assert NZC * ZR == RPT and REM <= ZR

    scratch = [
        pltpu.VMEM((NCH, CH), jnp.int32),
        pltpu.VMEM((NCH, CH), jnp.int32),
        pltpu.VMEM((NBUF, CH, d), jnp.float32),
        pltpu.VMEM((ZR, d), jnp.float32),
        pltpu.VMEM_SHARED((n, d), jnp.float32),
    ] + [pltpu.SemaphoreType.DMA] * (2 * NBUF + 2)

    @functools.partial(
        pl.kernel,
        out_type=jax.ShapeDtypeStruct((NC, n, d), jnp.float32),
        mesh=mesh,
        scratch_types=scratch,
        compiler_params=pltpu.CompilerParams(use_tc_tiling_on_sc=False),
    )
    def scatter_kernel(y_hbm, ei_hbm, out_hbm,
                       src_v, dst_v, bufs, zblk, acc, *sems):
        cid = lax.axis_index("c")
        sid = lax.axis_index("s")
        wid = cid * NS + sid
        gsem = sems[:NBUF]
        ssem = sems[NBUF:2 * NBUF]
        isem = sems[2 * NBUF]
        jsem = sems[2 * NBUF + 1]

        pltpu.async_copy(ei_hbm.at[0, wid], src_v, isem)
        pltpu.async_copy(ei_hbm.at[1, wid], dst_v, jsem)
        r0 = sid * RPT

        @pl.when(cid == 0)
        def _():
            pltpu.sync_copy(y_hbm.at[pl.ds(r0, RPT)], acc.at[pl.ds(r0, RPT)])
            if REM:
                @pl.when(sid == NS - 1)
                def _():
                    pltpu.sync_copy(y_hbm.at[pl.ds(RPT * NS, REM)],
                                    acc.at[pl.ds(RPT * NS, REM)])

        @pl.when(cid != 0)
        def _():
            zv = jnp.zeros((16,), jnp.float32)

            @pl.loop(0, ZR)
            def _zrow(rr):
                for cc in range(d // 16):
                    zblk[rr, pl.ds(cc * 16, 16)] = zv

            for k in range(NZC):
                pltpu.sync_copy(zblk, acc.at[pl.ds(r0 + k * ZR, ZR)])
            if REM:
                @pl.when(sid == NS - 1)
                def _():
                    pltpu.sync_copy(zblk.at[pl.ds(0, REM)],
                                    acc.at[pl.ds(RPT * NS, REM)])

        pltpu.make_async_copy(ei_hbm.at[0, wid], src_v, isem).wait()
        pltpu.make_async_copy(ei_hbm.at[1, wid], dst_v, jsem).wait()
        plsc.subcore_barrier()

        for b in range(NBUF):
            pltpu.async_copy(y_hbm.at[src_v.at[b]], bufs.at[b], gsem[b])

        LAG = 2

        @pl.loop(0, ROUNDS)
        def _round(r):
            for b in range(NBUF):
                j = r * NBUF + b
                pltpu.make_async_copy(
                    y_hbm.at[src_v.at[j]], bufs.at[b], gsem[b]).wait()
                pltpu.async_copy(
                    bufs.at[b], acc.at[dst_v.at[j]], ssem[b], add=True)
                bb = (b - LAG) % NBUF
                jj = j - LAG

                @pl.when(jnp.logical_and(jj >= 0, jj + NBUF < NCH))
                def _():
                    pltpu.make_async_copy(
                        bufs.at[bb], acc.at[dst_v.at[jj]], ssem[bb]).wait()
                    pltpu.async_copy(
                        y_hbm.at[src_v.at[jj + NBUF]], bufs.at[bb], gsem[bb])

        for b in range(NBUF):
            jj = NCH - NBUF + b
            pltpu.make_async_copy(
                bufs.at[b], acc.at[dst_v.at[jj]], ssem[b]).wait()
        plsc.subcore_barrier()
        pltpu.sync_copy(acc.at[pl.ds(r0, RPT)],
                        out_hbm.at[cid, pl.ds(r0, RPT)])
        if REM:
            @pl.when(sid == NS - 1)
            def _():
                pltpu.sync_copy(acc.at[pl.ds(RPT * NS, REM)],
                                out_hbm.at[cid, pl.ds(RPT * NS, REM)])

    return scatter_kernel


_TC_PARAMS = pltpu.CompilerParams(vmem_limit_bytes=100 * 1024 * 1024)


def _mm_body(h_ref, w_ref, o_ref):
    o_ref[...] = jnp.dot(h_ref[...], w_ref[...],
                         preferred_element_type=jnp.float32)


def _mm(h, w):
    n, din = h.shape
    dout = w.shape[1]
    return pl.pallas_call(
        _mm_body,
        out_shape=jax.ShapeDtypeStruct((n, dout), jnp.float32),
        compiler_params=_TC_PARAMS,
    )(h, w)


def _rep(v_ref, pk):
    v = v_ref[...]
    return jnp.concatenate([v] * pk, axis=1) if pk > 1 else v


def _bn_relu_packed(z, g_ref, bt_ref, n, pk, d):
    s = jnp.sum(z, axis=0, keepdims=True)
    s2 = jnp.sum(z * z, axis=0, keepdims=True)
    sf = s[:, 0:d]
    s2f = s2[:, 0:d]
    for k in range(1, pk):
        sf = sf + s[:, k * d:(k + 1) * d]
        s2f = s2f + s2[:, k * d:(k + 1) * d]
    mu = sf * (1.0 / n)
    var = s2f * (1.0 / n) - mu * mu
    rstd = lax.rsqrt(var + 1e-5)
    mur = jnp.concatenate([mu] * pk, axis=1)
    rstdr = jnp.concatenate([rstd] * pk, axis=1)
    return jnp.maximum((z - mur) * (rstdr * _rep(g_ref, pk)) +
                       _rep(bt_ref, pk), 0.0)


def _packed_mm(r, w_ref, pk, d):
    w = w_ref[...]
    outs = [jnp.dot(r[:, s * d:(s + 1) * d], w,
                    preferred_element_type=jnp.float32) for s in range(pk)]
    return jnp.concatenate(outs, axis=1) if pk > 1 else outs[0]


def _seg_body(p_ref, b1_ref, g1_ref, bt1_ref, w2_ref, b2_ref, g2_ref,
              bt2_ref, wn_ref, o_ref, *, n, pk, d):
    q = p_ref[0] + p_ref[1] + _rep(b1_ref, pk)
    r = _bn_relu_packed(q, g1_ref, bt1_ref, n, pk, d)
    t = _packed_mm(r, w2_ref, pk, d) + _rep(b2_ref, pk)
    r2 = _bn_relu_packed(t, g2_ref, bt2_ref, n, pk, d)
    o_ref[...] = _packed_mm(r2, wn_ref, pk, d)


def _segment(p, b1, g1, bt1, w2, b2, g2, bt2, wn, n, pk, d):
    m = p.shape[1]
    dout = pk * wn.shape[1]
    return pl.pallas_call(
        functools.partial(_seg_body, n=n, pk=pk, d=d),
        out_shape=jax.ShapeDtypeStruct((m, dout), jnp.float32),
        compiler_params=_TC_PARAMS,
    )(p, b1.reshape(1, d), g1.reshape(1, d), bt1.reshape(1, d), w2,
      b2.reshape(1, d), g2.reshape(1, d), bt2.reshape(1, d), wn)


def _final_body(p_ref, b1_ref, g1_ref, bt1_ref, w2_ref, b2_ref, g2_ref,
                bt2_ref, seg_ref, wfc_ref, bfc_ref, o_ref, *, n, pk, d):
    q = p_ref[0] + p_ref[1] + _rep(b1_ref, pk)
    r = _bn_relu_packed(q, g1_ref, bt1_ref, n, pk, d)
    t = _packed_mm(r, w2_ref, pk, d) + _rep(b2_ref, pk)
    r2 = _bn_relu_packed(t, g2_ref, bt2_ref, n, pk, d)
    m = r2.shape[0]
    ids = lax.broadcasted_iota(jnp.int32, (m, NGRAPHS), 1)
    ones = jnp.ones((m, 1), jnp.float32)
    accT = jnp.zeros((d + 1, NGRAPHS), jnp.float32)
    for s in range(pk):
        ohT = (seg_ref[:, s:s + 1] == ids).astype(jnp.float32)
        ra = jnp.concatenate([r2[:, s * d:(s + 1) * d], ones], axis=1)
        accT = accT + lax.dot_general(
            ra, ohT, (((0,), (0,)), ((), ())),
            preferred_element_type=jnp.float32)
    pooledT = accT[:d, :]
    cntT = accT[d:d + 1, :]
    meanT = pooledT / jnp.maximum(cntT, 1.0)
    o_ref[...] = lax.dot_general(
        meanT, wfc_ref[...], (((0,), (0,)), ((), ())),
        preferred_element_type=jnp.float32) + bfc_ref[...]


def _final(p, b1, g1, bt1, w2, b2, g2, bt2, segT, wfc, bfc, n, pk, d):
    ncls = wfc.shape[1]
    return pl.pallas_call(
        functools.partial(_final_body, n=n, pk=pk, d=d),
        out_shape=jax.ShapeDtypeStruct((NGRAPHS, ncls), jnp.float32),
        compiler_params=_TC_PARAMS,
    )(p, b1.reshape(1, d), g1.reshape(1, d), bt1.reshape(1, d), w2,
      b2.reshape(1, d), g2.reshape(1, d), bt2.reshape(1, d), segT,
      wfc, bfc.reshape(1, ncls))


def kernel(x, edge_index, batch,
           W1_1, b1_1, g1_1, bt1_1, W1_2, b1_2, g1_2, bt1_2,
           W2_1, b2_1, g2_1, bt2_1, W2_2, b2_2, g2_2, bt2_2,
           W3_1, b3_1, g3_1, bt3_1, W3_2, b3_2, g3_2, bt3_2,
           Wfc, bfc):
    n = x.shape[0]
    e = edge_index.shape[1]

    NW = NC * NS
    EPW = e // NW
    CH = 125
    NCH = EPW // CH
    ei3 = edge_index.reshape(2, NW, NCH, CH)
    batch4 = batch.reshape(n // 4, 4)

    scat64 = _make_scatter(n, e, 64)
    scat32 = _make_scatter(n, e, 32)

    y = _mm(x, W1_1)
    p = scat64(y, ei3)
    y = _segment(p.reshape(2, n // 2, 128), b1_1, g1_1, bt1_1, W1_2,
                 b1_2, g1_2, bt1_2, W2_1, n, 2, 64)
    p = scat64(y.reshape(n, 64), ei3)
    y = _segment(p.reshape(2, n // 2, 128), b2_1, g2_1, bt2_1, W2_2,
                 b2_2, g2_2, bt2_2, W3_1, n, 2, 64)
    p = scat32(y.reshape(n, 32), ei3)
    return _final(p.reshape(2, n // 4, 128), b3_1, g3_1, bt3_1, W3_2,
                  b3_2, g3_2, bt3_2, batch4, Wfc, bfc, n, 4, 32)


# speedup vs baseline: 1.0073x; 1.0073x over previous
import functools

import jax
import jax.numpy as jnp
from jax import lax
from jax.experimental import pallas as pl
from jax.experimental.pallas import tpu as pltpu
from jax.experimental.pallas import tpu_sc as plsc

NC = 2
NS = 16
NGRAPHS = 64


@functools.cache
def _make_scatter(n, e, d):
    NW = NC * NS
    EPW = e // NW
    CH = 125
    NCH = EPW // CH
    NBUF = 8
    ROUNDS = NCH // NBUF
    assert EPW * NW == e and CH * NCH == EPW and NBUF * ROUNDS == NCH
    RPT = (n // NS) // 8 * 8
    REM = n - RPT * NS
    assert RPT % 8 == 0 and REM % 8 == 0

    mesh = plsc.VectorSubcoreMesh(
        core_axis_name="c", subcore_axis_name="s",
        num_cores=NC, num_subcores=NS)
    ZR = 208
    NZC = RPT // ZR
    assert NZC * ZR == RPT and REM <= ZR

    scratch = [
        pltpu.VMEM((NCH, CH), jnp.int32),
        pltpu.VMEM((NCH, CH), jnp.int32),
        pltpu.VMEM((NBUF, CH, d), jnp.float32),
        pltpu.VMEM((ZR, d), jnp.float32),
        pltpu.VMEM_SHARED((n, d), jnp.float32),
    ] + [pltpu.SemaphoreType.DMA] * (2 * NBUF + 2)

    @functools.partial(
        pl.kernel,
        out_type=jax.ShapeDtypeStruct((NC, n, d), jnp.float32),
        mesh=mesh,
        scratch_types=scratch,
        compiler_params=pltpu.CompilerParams(use_tc_tiling_on_sc=False),
    )
    def scatter_kernel(y_hbm, ei_hbm, out_hbm,
                       src_v, dst_v, bufs, zblk, acc, *sems):
        cid = lax.axis_index("c")
        sid = lax.axis_index("s")
        wid = cid * NS + sid
        gsem = sems[:NBUF]
        ssem = sems[NBUF:2 * NBUF]
        isem = sems[2 * NBUF]
        jsem = sems[2 * NBUF + 1]

        pltpu.async_copy(ei_hbm.at[0, wid], src_v, isem)
        pltpu.async_copy(ei_hbm.at[1, wid], dst_v, jsem)
        r0 = sid * RPT

        @pl.when(cid == 0)
        def _():
            pltpu.sync_copy(y_hbm.at[pl.ds(r0, RPT)], acc.at[pl.ds(r0, RPT)])
            if REM:
                @pl.when(sid == NS - 1)
                def _():
                    pltpu.sync_copy(y_hbm.at[pl.ds(RPT * NS, REM)],
                                    acc.at[pl.ds(RPT * NS, REM)])

        @pl.when(cid != 0)
        def _():
            zv = jnp.zeros((16,), jnp.float32)

            @pl.loop(0, ZR)
            def _zrow(rr):
                for cc in range(d // 16):
                    zblk[rr, pl.ds(cc * 16, 16)] = zv

            for k in range(NZC):
                pltpu.sync_copy(zblk, acc.at[pl.ds(r0 + k * ZR, ZR)])
            if REM:
                @pl.when(sid == NS - 1)
                def _():
                    pltpu.sync_copy(zblk.at[pl.ds(0, REM)],
                                    acc.at[pl.ds(RPT * NS, REM)])

        pltpu.make_async_copy(ei_hbm.at[0, wid], src_v, isem).wait()
        pltpu.make_async_copy(ei_hbm.at[1, wid], dst_v, jsem).wait()
        plsc.subcore_barrier()

        for b in range(NBUF):
            pltpu.async_copy(y_hbm.at[src_v.at[b]], bufs.at[b], gsem[b])

        LAG = 2

        @pl.loop(0, ROUNDS)
        def _round(r):
            for b in range(NBUF):
                j = r * NBUF + b
                pltpu.make_async_copy(
                    y_hbm.at[src_v.at[j]], bufs.at[b], gsem[b]).wait()
                pltpu.async_copy(
                    bufs.at[b], acc.at[dst_v.at[j]], ssem[b], add=True)
                bb = (b - LAG) % NBUF
                jj = j - LAG

                @pl.when(jnp.logical_and(jj >= 0, jj + NBUF < NCH))
                def _():
                    pltpu.make_async_copy(
                        bufs.at[bb], acc.at[dst_v.at[jj]], ssem[bb]).wait()
                    pltpu.async_copy(
                        y_hbm.at[src_v.at[jj + NBUF]], bufs.at[bb], gsem[bb])

        for b in range(NBUF):
            jj = NCH - NBUF + b
            pltpu.make_async_copy(
                bufs.at[b], acc.at[dst_v.at[jj]], ssem[b]).wait()
        plsc.subcore_barrier()
        pltpu.sync_copy(acc.at[pl.ds(r0, RPT)],
                        out_hbm.at[cid, pl.ds(r0, RPT)])
        if REM:
            @pl.when(sid == NS - 1)
            def _():
                pltpu.sync_copy(acc.at[pl.ds(RPT * NS, REM)],
                                out_hbm.at[cid, pl.ds(RPT * NS, REM)])

    return scatter_kernel


_TC_PARAMS = pltpu.CompilerParams(vmem_limit_bytes=100 * 1024 * 1024)


def _mm_body(h_ref, w_ref, o_ref):
    o_ref[...] = jnp.dot(h_ref[...], w_ref[...],
                         preferred_element_type=jnp.float32)


def _mm(h, w):
    n, din = h.shape
    dout = w.shape[1]
    bm = 2000
    return pl.pallas_call(
        _mm_body,
        grid=(n // bm,),
        in_specs=[pl.BlockSpec((bm, din), lambda i: (i, 0)),
                  pl.BlockSpec((din, dout), lambda i: (0, 0))],
        out_specs=pl.BlockSpec((bm, dout), lambda i: (i, 0)),
        out_shape=jax.ShapeDtypeStruct((n, dout), jnp.float32),
        compiler_params=_TC_PARAMS,
    )(h, w)


def _rep(v_ref, pk):
    v = v_ref[...]
    return jnp.concatenate([v] * pk, axis=1) if pk > 1 else v


def _bn_relu_packed(z, g_ref, bt_ref, n, pk, d):
    s = jnp.sum(z, axis=0, keepdims=True)
    s2 = jnp.sum(z * z, axis=0, keepdims=True)
    sf = s[:, 0:d]
    s2f = s2[:, 0:d]
    for k in range(1, pk):
        sf = sf + s[:, k * d:(k + 1) * d]
        s2f = s2f + s2[:, k * d:(k + 1) * d]
    mu = sf * (1.0 / n)
    var = s2f * (1.0 / n) - mu * mu
    rstd = lax.rsqrt(var + 1e-5)
    mur = jnp.concatenate([mu] * pk, axis=1)
    rstdr = jnp.concatenate([rstd] * pk, axis=1)
    return jnp.maximum((z - mur) * (rstdr * _rep(g_ref, pk)) +
                       _rep(bt_ref, pk), 0.0)


def _packed_mm(r, w_ref, pk, d):
    w = w_ref[...]
    outs = [jnp.dot(r[:, s * d:(s + 1) * d], w,
                    preferred_element_type=jnp.float32) for s in range(pk)]
    return jnp.concatenate(outs, axis=1) if pk > 1 else outs[0]


def _seg_body(p_ref, b1_ref, g1_ref, bt1_ref, w2_ref, b2_ref, g2_ref,
              bt2_ref, wn_ref, o_ref, *, n, pk, d):
    q = p_ref[0] + p_ref[1] + _rep(b1_ref, pk)
    r = _bn_relu_packed(q, g1_ref, bt1_ref, n, pk, d)
    t = _packed_mm(r, w2_ref, pk, d) + _rep(b2_ref, pk)
    r2 = _bn_relu_packed(t, g2_ref, bt2_ref, n, pk, d)
    o_ref[...] = _packed_mm(r2, wn_ref, pk, d)


def _segment(p, b1, g1, bt1, w2, b2, g2, bt2, wn, n, pk, d):
    m = p.shape[1]
    dout = pk * wn.shape[1]
    return pl.pallas_call(
        functools.partial(_seg_body, n=n, pk=pk, d=d),
        out_shape=jax.ShapeDtypeStruct((m, dout), jnp.float32),
        compiler_params=_TC_PARAMS,
    )(p, b1.reshape(1, d), g1.reshape(1, d), bt1.reshape(1, d), w2,
      b2.reshape(1, d), g2.reshape(1, d), bt2.reshape(1, d), wn)


def _final_body(p_ref, b1_ref, g1_ref, bt1_ref, w2_ref, b2_ref, g2_ref,
                bt2_ref, seg_ref, wfc_ref, bfc_ref, o_ref, *, n, pk, d):
    q = p_ref[0] + p_ref[1] + _rep(b1_ref, pk)
    r = _bn_relu_packed(q, g1_ref, bt1_ref, n, pk, d)
    t = _packed_mm(r, w2_ref, pk, d) + _rep(b2_ref, pk)
    r2 = _bn_relu_packed(t, g2_ref, bt2_ref, n, pk, d)
    m = r2.shape[0]
    ids = lax.broadcasted_iota(jnp.int32, (NGRAPHS, m), 0)
    ones = jnp.ones((m, 1), jnp.float32)
    acc = jnp.zeros((NGRAPHS, d + 1), jnp.float32)
    for s in range(pk):
        oh = (seg_ref[s, :][None, :] == ids).astype(jnp.float32)
        ra = jnp.concatenate([r2[:, s * d:(s + 1) * d], ones], axis=1)
        acc = acc + jnp.dot(oh, ra, preferred_element_type=jnp.float32)
    pooled = acc[:, :d]
    cnt = acc[:, d:d + 1]
    mean = pooled / jnp.maximum(cnt, 1.0)
    o_ref[...] = jnp.dot(mean, wfc_ref[...],
                         preferred_element_type=jnp.float32) + bfc_ref[...]


def _final(p, b1, g1, bt1, w2, b2, g2, bt2, segT, wfc, bfc, n, pk, d):
    ncls = wfc.shape[1]
    return pl.pallas_call(
        functools.partial(_final_body, n=n, pk=pk, d=d),
        out_shape=jax.ShapeDtypeStruct((NGRAPHS, ncls), jnp.float32),
        compiler_params=_TC_PARAMS,
    )(p, b1.reshape(1, d), g1.reshape(1, d), bt1.reshape(1, d), w2,
      b2.reshape(1, d), g2.reshape(1, d), bt2.reshape(1, d), segT,
      wfc, bfc.reshape(1, ncls))


def kernel(x, edge_index, batch,
           W1_1, b1_1, g1_1, bt1_1, W1_2, b1_2, g1_2, bt1_2,
           W2_1, b2_1, g2_1, bt2_1, W2_2, b2_2, g2_2, bt2_2,
           W3_1, b3_1, g3_1, bt3_1, W3_2, b3_2, g3_2, bt3_2,
           Wfc, bfc):
    n = x.shape[0]
    e = edge_index.shape[1]

    NW = NC * NS
    EPW = e // NW
    CH = 125
    NCH = EPW // CH
    ei3 = edge_index.reshape(2, NW, NCH, CH)
    segT = batch.reshape(n // 4, 4).T

    scat64 = _make_scatter(n, e, 64)
    scat32 = _make_scatter(n, e, 32)

    y = _mm(x, W1_1)
    p = scat64(y, ei3)
    y = _segment(p.reshape(2, n // 2, 128), b1_1, g1_1, bt1_1, W1_2,
                 b1_2, g1_2, bt1_2, W2_1, n, 2, 64)
    p = scat64(y.reshape(n, 64), ei3)
    y = _segment(p.reshape(2, n // 2, 128), b2_1, g2_1, bt2_1, W2_2,
                 b2_2, g2_2, bt2_2, W3_1, n, 2, 64)
    p = scat32(y.reshape(n, 32), ei3)
    return _final(p.reshape(2, n // 4, 128), b3_1, g3_1, bt3_1, W3_2,
                  b3_2, g3_2, bt3_2, segT, Wfc, bfc, n, 4, 32)


# speedup vs baseline: 1.0132x; 1.0059x over previous
import functools

import jax
import jax.numpy as jnp
from jax import lax
from jax.experimental import pallas as pl
from jax.experimental.pallas import tpu as pltpu
from jax.experimental.pallas import tpu_sc as plsc

NC = 2
NS = 16
NGRAPHS = 64


@functools.cache
def _make_scatter(n, e, d):
    NW = NC * NS
    EPW = e // NW
    CH = 125
    NCH = EPW // CH
    NBUF = 8
    ROUNDS = NCH // NBUF
    assert EPW * NW == e and CH * NCH == EPW and NBUF * ROUNDS == NCH
    RPT = (n // NS) // 8 * 8
    REM = n - RPT * NS
    assert RPT % 8 == 0 and REM % 8 == 0

    mesh = plsc.VectorSubcoreMesh(
        core_axis_name="c", subcore_axis_name="s",
        num_cores=NC, num_subcores=NS)
    ZR = 208
    NZC = RPT // ZR
    assert NZC * ZR == RPT and REM <= ZR

    scratch = [
        pltpu.VMEM((NCH, CH), jnp.int32),
        pltpu.VMEM((NCH, CH), jnp.int32),
        pltpu.VMEM((NBUF, CH, d), jnp.float32),
        pltpu.VMEM((ZR, d), jnp.float32),
        pltpu.VMEM_SHARED((n, d), jnp.float32),
    ] + [pltpu.SemaphoreType.DMA] * (2 * NBUF + 2)

    @functools.partial(
        pl.kernel,
        out_type=jax.ShapeDtypeStruct((NC, n, d), jnp.float32),
        mesh=mesh,
        scratch_types=scratch,
        compiler_params=pltpu.CompilerParams(use_tc_tiling_on_sc=False),
    )
    def scatter_kernel(y_hbm, ei_hbm, out_hbm,
                       src_v, dst_v, bufs, zblk, acc, *sems):
        cid = lax.axis_index("c")
        sid = lax.axis_index("s")
        wid = cid * NS + sid
        gsem = sems[:NBUF]
        ssem = sems[NBUF:2 * NBUF]
        isem = sems[2 * NBUF]
        jsem = sems[2 * NBUF + 1]

        pltpu.async_copy(ei_hbm.at[0, wid], src_v, isem)
        pltpu.async_copy(ei_hbm.at[1, wid], dst_v, jsem)
        r0 = sid * RPT

        @pl.when(cid == 0)
        def _():
            pltpu.sync_copy(y_hbm.at[pl.ds(r0, RPT)], acc.at[pl.ds(r0, RPT)])
            if REM:
                @pl.when(sid == NS - 1)
                def _():
                    pltpu.sync_copy(y_hbm.at[pl.ds(RPT * NS, REM)],
                                    acc.at[pl.ds(RPT * NS, REM)])

        @pl.when(cid != 0)
        def _():
            zv = jnp.zeros((16,), jnp.float32)

            @pl.loop(0, ZR)
            def _zrow(rr):
                for cc in range(d // 16):
                    zblk[rr, pl.ds(cc * 16, 16)] = zv

            for k in range(NZC):
                pltpu.sync_copy(zblk, acc.at[pl.ds(r0 + k * ZR, ZR)])
            if REM:
                @pl.when(sid == NS - 1)
                def _():
                    pltpu.sync_copy(zblk.at[pl.ds(0, REM)],
                                    acc.at[pl.ds(RPT * NS, REM)])

        pltpu.make_async_copy(ei_hbm.at[0, wid], src_v, isem).wait()
        pltpu.make_async_copy(ei_hbm.at[1, wid], dst_v, jsem).wait()
        plsc.subcore_barrier()

        for b in range(NBUF):
            pltpu.async_copy(y_hbm.at[src_v.at[b]], bufs.at[b], gsem[b])

        LAG = 2

        @pl.loop(0, ROUNDS)
        def _round(r):
            for b in range(NBUF):
                j = r * NBUF + b
                pltpu.make_async_copy(
                    y_hbm.at[src_v.at[j]], bufs.at[b], gsem[b]).wait()
                pltpu.async_copy(
                    bufs.at[b], acc.at[dst_v.at[j]], ssem[b], add=True)
                bb = (b - LAG) % NBUF
                jj = j - LAG

                @pl.when(jnp.logical_and(jj >= 0, jj + NBUF < NCH))
                def _():
                    pltpu.make_async_copy(
                        bufs.at[bb], acc.at[dst_v.at[jj]], ssem[bb]).wait()
                    pltpu.async_copy(
                        y_hbm.at[src_v.at[jj + NBUF]], bufs.at[bb], gsem[bb])

        for b in range(NBUF):
            jj = NCH - NBUF + b
            pltpu.make_async_copy(
                bufs.at[b], acc.at[dst_v.at[jj]], ssem[b]).wait()
        plsc.subcore_barrier()
        pltpu.sync_copy(acc.at[pl.ds(r0, RPT)],
                        out_hbm.at[cid, pl.ds(r0, RPT)])
        if REM:
            @pl.when(sid == NS - 1)
            def _():
                pltpu.sync_copy(acc.at[pl.ds(RPT * NS, REM)],
                                out_hbm.at[cid, pl.ds(RPT * NS, REM)])

    return scatter_kernel


_TC_PARAMS = pltpu.CompilerParams(vmem_limit_bytes=100 * 1024 * 1024)


def _mm_body(h_ref, w_ref, o_ref):
    o_ref[...] = jnp.dot(h_ref[...], w_ref[...],
                         preferred_element_type=jnp.float32)


def _mm(h, w):
    n, din = h.shape
    dout = w.shape[1]
    return pl.pallas_call(
        _mm_body,
        out_shape=jax.ShapeDtypeStruct((n, dout), jnp.float32),
        compiler_params=_TC_PARAMS,
    )(h, w)


def _rep(v_ref, pk):
    v = v_ref[...]
    return jnp.concatenate([v] * pk, axis=1) if pk > 1 else v


def _bn_relu_packed(z, g_ref, bt_ref, n, pk, d):
    s = jnp.sum(z, axis=0, keepdims=True)
    s2 = jnp.sum(z * z, axis=0, keepdims=True)
    sf = s[:, 0:d]
    s2f = s2[:, 0:d]
    for k in range(1, pk):
        sf = sf + s[:, k * d:(k + 1) * d]
        s2f = s2f + s2[:, k * d:(k + 1) * d]
    mu = sf * (1.0 / n)
    var = s2f * (1.0 / n) - mu * mu
    rstd = lax.rsqrt(var + 1e-5)
    mur = jnp.concatenate([mu] * pk, axis=1)
    rstdr = jnp.concatenate([rstd] * pk, axis=1)
    return jnp.maximum((z - mur) * (rstdr * _rep(g_ref, pk)) +
                       _rep(bt_ref, pk), 0.0)


def _packed_mm(r, w_ref, pk, d):
    w = w_ref[...]
    outs = [jnp.dot(r[:, s * d:(s + 1) * d], w,
                    preferred_element_type=jnp.float32) for s in range(pk)]
    return jnp.concatenate(outs, axis=1) if pk > 1 else outs[0]


def _seg_body(p_ref, b1_ref, g1_ref, bt1_ref, w2_ref, b2_ref, g2_ref,
              bt2_ref, wn_ref, o_ref, *, n, pk, d):
    q = p_ref[0] + p_ref[1] + _rep(b1_ref, pk)
    r = _bn_relu_packed(q, g1_ref, bt1_ref, n, pk, d)
    t = _packed_mm(r, w2_ref, pk, d) + _rep(b2_ref, pk)
    r2 = _bn_relu_packed(t, g2_ref, bt2_ref, n, pk, d)
    o_ref[...] = _packed_mm(r2, wn_ref, pk, d)


def _segment(p, b1, g1, bt1, w2, b2, g2, bt2, wn, n, pk, d):
    m = p.shape[1]
    dout = pk * wn.shape[1]
    return pl.pallas_call(
        functools.partial(_seg_body, n=n, pk=pk, d=d),
        out_shape=jax.ShapeDtypeStruct((m, dout), jnp.float32),
        compiler_params=_TC_PARAMS,
    )(p, b1.reshape(1, d), g1.reshape(1, d), bt1.reshape(1, d), w2,
      b2.reshape(1, d), g2.reshape(1, d), bt2.reshape(1, d), wn)


def _final_body(p_ref, b1_ref, g1_ref, bt1_ref, w2_ref, b2_ref, g2_ref,
                bt2_ref, seg_ref, wfc_ref, bfc_ref, o_ref, *, n, pk, d):
    q = p_ref[0] + p_ref[1] + _rep(b1_ref, pk)
    r = _bn_relu_packed(q, g1_ref, bt1_ref, n, pk, d)
    t = _packed_mm(r, w2_ref, pk, d) + _rep(b2_ref, pk)
    r2 = _bn_relu_packed(t, g2_ref, bt2_ref, n, pk, d)
    m = r2.shape[0]
    ids = lax.broadcasted_iota(jnp.int32, (NGRAPHS, m), 0)
    ones = jnp.ones((m, 1), jnp.float32)
    acc = jnp.zeros((NGRAPHS, d + 1), jnp.float32)
    for s in range(pk):
        oh = (seg_ref[s, :][None, :] == ids).astype(jnp.float32)
        ra = jnp.concatenate([r2[:, s * d:(s + 1) * d], ones], axis=1)
        acc = acc + jnp.dot(oh, ra, preferred_element_type=jnp.float32)
    pooled = acc[:, :d]
    cnt = acc[:, d:d + 1]
    mean = pooled / jnp.maximum(cnt, 1.0)
    o_ref[...] = jnp.dot(mean, wfc_ref[...],
                         preferred_element_type=jnp.float32) + bfc_ref[...]


def _final(p, b1, g1, bt1, w2, b2, g2, bt2, segT, wfc, bfc, n, pk, d):
    ncls = wfc.shape[1]
    return pl.pallas_call(
        functools.partial(_final_body, n=n, pk=pk, d=d),
        out_shape=jax.ShapeDtypeStruct((NGRAPHS, ncls), jnp.float32),
        compiler_params=_TC_PARAMS,
    )(p, b1.reshape(1, d), g1.reshape(1, d), bt1.reshape(1, d), w2,
      b2.reshape(1, d), g2.reshape(1, d), bt2.reshape(1, d), segT,
      wfc, bfc.reshape(1, ncls))


def kernel(x, edge_index, batch,
           W1_1, b1_1, g1_1, bt1_1, W1_2, b1_2, g1_2, bt1_2,
           W2_1, b2_1, g2_1, bt2_1, W2_2, b2_2, g2_2, bt2_2,
           W3_1, b3_1, g3_1, bt3_1, W3_2, b3_2, g3_2, bt3_2,
           Wfc, bfc):
    n = x.shape[0]
    e = edge_index.shape[1]

    NW = NC * NS
    EPW = e // NW
    CH = 125
    NCH = EPW // CH
    ei3 = edge_index.reshape(2, NW, NCH, CH)
    segT = batch.reshape(n // 4, 4).T

    scat64 = _make_scatter(n, e, 64)
    scat32 = _make_scatter(n, e, 32)

    y = _mm(x, W1_1)
    p = scat64(y, ei3)
    y = _segment(p.reshape(2, n // 2, 128), b1_1, g1_1, bt1_1, W1_2,
                 b1_2, g1_2, bt1_2, W2_1, n, 2, 64)
    p = scat64(y.reshape(n, 64), ei3)
    y = _segment(p.reshape(2, n // 2, 128), b2_1, g2_1, bt2_1, W2_2,
                 b2_2, g2_2, bt2_2, W3_1, n, 2, 64)
    p = scat32(y.reshape(n, 32), ei3)
    return _final(p.reshape(2, n // 4, 128), b3_1, g3_1, bt3_1, W3_2,
                  b3_2, g3_2, bt3_2, segT, Wfc, bfc, n, 4, 32)


# speedup vs baseline: 1.0260x; 1.0126x over previous
import functools

import jax
import jax.numpy as jnp
from jax import lax
from jax.experimental import pallas as pl
from jax.experimental.pallas import tpu as pltpu
from jax.experimental.pallas import tpu_sc as plsc

NC = 2
NS = 16
NGRAPHS = 64


@functools.cache
def _make_scatter(n, e, d):
    NW = NC * NS
    EPW = e // NW
    CH = 100
    NCH = EPW // CH
    NBUF = 10
    ROUNDS = NCH // NBUF
    assert EPW * NW == e and CH * NCH == EPW and NBUF * ROUNDS == NCH
    RPT = (n // NS) // 8 * 8
    REM = n - RPT * NS
    assert RPT % 8 == 0 and REM % 8 == 0

    mesh = plsc.VectorSubcoreMesh(
        core_axis_name="c", subcore_axis_name="s",
        num_cores=NC, num_subcores=NS)
    ZR = 208
    NZC = RPT // ZR
    assert NZC * ZR == RPT and REM <= ZR

    scratch = [
        pltpu.VMEM((NCH, CH), jnp.int32),
        pltpu.VMEM((NCH, CH), jnp.int32),
        pltpu.VMEM((NBUF, CH, d), jnp.float32),
        pltpu.VMEM((ZR, d), jnp.float32),
        pltpu.VMEM_SHARED((n, d), jnp.float32),
    ] + [pltpu.SemaphoreType.DMA] * (2 * NBUF + 2)

    @functools.partial(
        pl.kernel,
        out_type=jax.ShapeDtypeStruct((NC, n, d), jnp.float32),
        mesh=mesh,
        scratch_types=scratch,
        compiler_params=pltpu.CompilerParams(use_tc_tiling_on_sc=False),
    )
    def scatter_kernel(y_hbm, ei_hbm, out_hbm,
                       src_v, dst_v, bufs, zblk, acc, *sems):
        cid = lax.axis_index("c")
        sid = lax.axis_index("s")
        wid = cid * NS + sid
        gsem = sems[:NBUF]
        ssem = sems[NBUF:2 * NBUF]
        isem = sems[2 * NBUF]
        jsem = sems[2 * NBUF + 1]

        pltpu.async_copy(ei_hbm.at[0, wid], src_v, isem)
        pltpu.async_copy(ei_hbm.at[1, wid], dst_v, jsem)
        r0 = sid * RPT

        @pl.when(cid == 0)
        def _():
            pltpu.sync_copy(y_hbm.at[pl.ds(r0, RPT)], acc.at[pl.ds(r0, RPT)])
            if REM:
                @pl.when(sid == NS - 1)
                def _():
                    pltpu.sync_copy(y_hbm.at[pl.ds(RPT * NS, REM)],
                                    acc.at[pl.ds(RPT * NS, REM)])

        @pl.when(cid != 0)
        def _():
            zv = jnp.zeros((16,), jnp.float32)

            @pl.loop(0, ZR)
            def _zrow(rr):
                for cc in range(d // 16):
                    zblk[rr, pl.ds(cc * 16, 16)] = zv

            for k in range(NZC):
                pltpu.sync_copy(zblk, acc.at[pl.ds(r0 + k * ZR, ZR)])
            if REM:
                @pl.when(sid == NS - 1)
                def _():
                    pltpu.sync_copy(zblk.at[pl.ds(0, REM)],
                                    acc.at[pl.ds(RPT * NS, REM)])

        pltpu.make_async_copy(ei_hbm.at[0, wid], src_v, isem).wait()
        pltpu.make_async_copy(ei_hbm.at[1, wid], dst_v, jsem).wait()
        plsc.subcore_barrier()

        for b in range(NBUF):
            pltpu.async_copy(y_hbm.at[src_v.at[b]], bufs.at[b], gsem[b])

        LAG = 2

        @pl.loop(0, ROUNDS)
        def _round(r):
            for b in range(NBUF):
                j = r * NBUF + b
                pltpu.make_async_copy(
                    y_hbm.at[src_v.at[j]], bufs.at[b], gsem[b]).wait()
                pltpu.async_copy(
                    bufs.at[b], acc.at[dst_v.at[j]], ssem[b], add=True)
                bb = (b - LAG) % NBUF
                jj = j - LAG

                @pl.when(jnp.logical_and(jj >= 0, jj + NBUF < NCH))
                def _():
                    pltpu.make_async_copy(
                        bufs.at[bb], acc.at[dst_v.at[jj]], ssem[bb]).wait()
                    pltpu.async_copy(
                        y_hbm.at[src_v.at[jj + NBUF]], bufs.at[bb], gsem[bb])

        for b in range(NBUF):
            jj = NCH - NBUF + b
            pltpu.make_async_copy(
                bufs.at[b], acc.at[dst_v.at[jj]], ssem[b]).wait()
        plsc.subcore_barrier()
        pltpu.sync_copy(acc.at[pl.ds(r0, RPT)],
                        out_hbm.at[cid, pl.ds(r0, RPT)])
        if REM:
            @pl.when(sid == NS - 1)
            def _():
                pltpu.sync_copy(acc.at[pl.ds(RPT * NS, REM)],
                                out_hbm.at[cid, pl.ds(RPT * NS, REM)])

    return scatter_kernel


_TC_PARAMS = pltpu.CompilerParams(vmem_limit_bytes=100 * 1024 * 1024)


def _mm_body(h_ref, w_ref, o_ref):
    o_ref[...] = jnp.dot(h_ref[...], w_ref[...],
                         preferred_element_type=jnp.float32)


def _mm(h, w):
    n, din = h.shape
    dout = w.shape[1]
    return pl.pallas_call(
        _mm_body,
        out_shape=jax.ShapeDtypeStruct((n, dout), jnp.float32),
        compiler_params=_TC_PARAMS,
    )(h, w)


def _rep(v_ref, pk):
    v = v_ref[...]
    return jnp.concatenate([v] * pk, axis=1) if pk > 1 else v


def _bn_relu_packed(z, g_ref, bt_ref, n, pk, d):
    s = jnp.sum(z, axis=0, keepdims=True)
    s2 = jnp.sum(z * z, axis=0, keepdims=True)
    sf = s[:, 0:d]
    s2f = s2[:, 0:d]
    for k in range(1, pk):
        sf = sf + s[:, k * d:(k + 1) * d]
        s2f = s2f + s2[:, k * d:(k + 1) * d]
    mu = sf * (1.0 / n)
    var = s2f * (1.0 / n) - mu * mu
    rstd = lax.rsqrt(var + 1e-5)
    mur = jnp.concatenate([mu] * pk, axis=1)
    rstdr = jnp.concatenate([rstd] * pk, axis=1)
    return jnp.maximum((z - mur) * (rstdr * _rep(g_ref, pk)) +
                       _rep(bt_ref, pk), 0.0)


def _packed_mm(r, w_ref, pk, d):
    w = w_ref[...]
    outs = [jnp.dot(r[:, s * d:(s + 1) * d], w,
                    preferred_element_type=jnp.float32) for s in range(pk)]
    return jnp.concatenate(outs, axis=1) if pk > 1 else outs[0]


def _seg_body(p_ref, b1_ref, g1_ref, bt1_ref, w2_ref, b2_ref, g2_ref,
              bt2_ref, wn_ref, o_ref, *, n, pk, d):
    q = p_ref[0] + p_ref[1] + _rep(b1_ref, pk)
    r = _bn_relu_packed(q, g1_ref, bt1_ref, n, pk, d)
    t = _packed_mm(r, w2_ref, pk, d) + _rep(b2_ref, pk)
    r2 = _bn_relu_packed(t, g2_ref, bt2_ref, n, pk, d)
    o_ref[...] = _packed_mm(r2, wn_ref, pk, d)


def _segment(p, b1, g1, bt1, w2, b2, g2, bt2, wn, n, pk, d):
    m = p.shape[1]
    dout = pk * wn.shape[1]
    return pl.pallas_call(
        functools.partial(_seg_body, n=n, pk=pk, d=d),
        out_shape=jax.ShapeDtypeStruct((m, dout), jnp.float32),
        compiler_params=_TC_PARAMS,
    )(p, b1.reshape(1, d), g1.reshape(1, d), bt1.reshape(1, d), w2,
      b2.reshape(1, d), g2.reshape(1, d), bt2.reshape(1, d), wn)


def _final_body(p_ref, b1_ref, g1_ref, bt1_ref, w2_ref, b2_ref, g2_ref,
                bt2_ref, seg_ref, wfc_ref, bfc_ref, o_ref, *, n, pk, d):
    q = p_ref[0] + p_ref[1] + _rep(b1_ref, pk)
    r = _bn_relu_packed(q, g1_ref, bt1_ref, n, pk, d)
    t = _packed_mm(r, w2_ref, pk, d) + _rep(b2_ref, pk)
    r2 = _bn_relu_packed(t, g2_ref, bt2_ref, n, pk, d)
    m = r2.shape[0]
    ids = lax.broadcasted_iota(jnp.int32, (NGRAPHS, m), 0)
    ones = jnp.ones((m, 1), jnp.float32)
    acc = jnp.zeros((NGRAPHS, d + 1), jnp.float32)
    for s in range(pk):
        oh = (seg_ref[s, :][None, :] == ids).astype(jnp.float32)
        ra = jnp.concatenate([r2[:, s * d:(s + 1) * d], ones], axis=1)
        acc = acc + jnp.dot(oh, ra, preferred_element_type=jnp.float32)
    pooled = acc[:, :d]
    cnt = acc[:, d:d + 1]
    mean = pooled / jnp.maximum(cnt, 1.0)
    o_ref[...] = jnp.dot(mean, wfc_ref[...],
                         preferred_element_type=jnp.float32) + bfc_ref[...]


def _final(p, b1, g1, bt1, w2, b2, g2, bt2, segT, wfc, bfc, n, pk, d):
    ncls = wfc.shape[1]
    return pl.pallas_call(
        functools.partial(_final_body, n=n, pk=pk, d=d),
        out_shape=jax.ShapeDtypeStruct((NGRAPHS, ncls), jnp.float32),
        compiler_params=_TC_PARAMS,
    )(p, b1.reshape(1, d), g1.reshape(1, d), bt1.reshape(1, d), w2,
      b2.reshape(1, d), g2.reshape(1, d), bt2.reshape(1, d), segT,
      wfc, bfc.reshape(1, ncls))


def kernel(x, edge_index, batch,
           W1_1, b1_1, g1_1, bt1_1, W1_2, b1_2, g1_2, bt1_2,
           W2_1, b2_1, g2_1, bt2_1, W2_2, b2_2, g2_2, bt2_2,
           W3_1, b3_1, g3_1, bt3_1, W3_2, b3_2, g3_2, bt3_2,
           Wfc, bfc):
    n = x.shape[0]
    e = edge_index.shape[1]

    NW = NC * NS
    EPW = e // NW
    CH = 100
    NCH = EPW // CH
    ei3 = edge_index.reshape(2, NW, NCH, CH)
    segT = batch.reshape(n // 4, 4).T

    scat64 = _make_scatter(n, e, 64)
    scat32 = _make_scatter(n, e, 32)

    y = _mm(x, W1_1)
    p = scat64(y, ei3)
    y = _segment(p.reshape(2, n // 2, 128), b1_1, g1_1, bt1_1, W1_2,
                 b1_2, g1_2, bt1_2, W2_1, n, 2, 64)
    p = scat64(y.reshape(n, 64), ei3)
    y = _segment(p.reshape(2, n // 2, 128), b2_1, g2_1, bt2_1, W2_2,
                 b2_2, g2_2, bt2_2, W3_1, n, 2, 64)
    p = scat32(y.reshape(n, 32), ei3)
    return _final(p.reshape(2, n // 4, 128), b3_1, g3_1, bt3_1, W3_2,
                  b3_2, g3_2, bt3_2, segT, Wfc, bfc, n, 4, 32)
